# baseline (speedup 1.0000x reference)
import functools

import jax
import jax.numpy as jnp
from jax import lax
from jax.experimental import pallas as pl
from jax.experimental.pallas import tpu as pltpu
from jax.experimental.pallas import tpu_sc as plsc

N = 10000
E = 320000
F = 128
NPAD = 10112
NW = 32
K = 72
G = 16
CH = 288
NG = CH // G
EPW = CH * K
E2P = NW * EPW
RPT = NPAD // 16
F32 = jnp.float32

_mesh = plsc.VectorSubcoreMesh(core_axis_name="c", subcore_axis_name="s")


@functools.partial(
    pl.kernel,
    out_type=jax.ShapeDtypeStruct((2, NPAD, F), F32),
    mesh=_mesh,
    scratch_types=[
        pltpu.VMEM((G, K), jnp.int32),
        pltpu.VMEM((G, K), jnp.int32),
        pltpu.VMEM((2, K, F), F32),
        pltpu.VMEM((2, K, F), F32),
        pltpu.VMEM((3 * G, K), F32),
        pltpu.VMEM((3, F), F32),
        pltpu.VMEM_SHARED((NPAD, F), F32),
        pltpu.SemaphoreType.DMA((2,)),
        pltpu.SemaphoreType.DMA((2,)),
    ],
)
def _edge_pass(ha, hb, th, wc, srcw, dstw, zz, out, idx_s, idx_d, buf_a,
               buf_b, buf_th, buf_w, acc, sem_i, sem_s):
    ci = lax.axis_index("c")
    si = lax.axis_index("s")
    w = si * 2 + ci
    r0 = si * RPT
    pltpu.sync_copy(zz.at[pl.ds(r0, RPT)], acc.at[pl.ds(r0, RPT)])
    pltpu.sync_copy(wc, buf_w)
    wv = [[buf_w[k, pl.ds(16 * v, 16)] for v in range(F // 16)]
          for k in range(3)]
    plsc.subcore_barrier()

    def issue_in(g, jj, slot):
        pltpu.async_copy(ha.at[idx_s.at[jj]], buf_a.at[slot], sem_i.at[slot])
        pltpu.async_copy(hb.at[idx_d.at[jj]], buf_b.at[slot], sem_i.at[slot])

    def wait_in(slot):
        for buf in (buf_a, buf_b):
            pltpu.make_async_copy(ha.at[pl.ds(0, K)], buf.at[slot],
                                  sem_i.at[slot]).wait()

    def wait_sc(slot):
        pltpu.make_async_copy(ha.at[pl.ds(0, K)], buf_b.at[slot],
                              sem_s.at[slot]).wait()

    def compute(jj, slot):
        def row(e, c2):
            e0 = (e // 16) * 16
            el = jnp.full((16,), e % 16, jnp.int32)
            dn = lax.GatherDimensionNumbers(offset_dims=(),
                                            collapsed_slice_dims=(0,),
                                            start_index_map=(0,))
            t0, t1, t2 = (
                lax.gather(buf_th[3 * jj + k, pl.ds(e0, 16)], el[:, None],
                           dimension_numbers=dn, slice_sizes=(1,),
                           mode=lax.GatherScatterMode.PROMISE_IN_BOUNDS)
                for k in range(3))
            for v in range(F // 16):
                sl = pl.ds(16 * v, 16)
                cv = t0 * wv[0][v] + t1 * wv[1][v] + t2 * wv[2][v]
                buf_a[slot, e, sl] = jnp.maximum(
                    buf_a[slot, e, sl] + buf_b[slot, e, sl] + cv, 0.0)
            return c2

        lax.fori_loop(0, K, row, 0)

    def group(g, carry):
        pltpu.sync_copy(srcw.at[w, pl.ds(g * G, G)], idx_s)
        pltpu.sync_copy(dstw.at[w, pl.ds(g * G, G)], idx_d)
        pltpu.sync_copy(th.at[w, pl.ds(g * 3 * G, 3 * G)], buf_th)
        issue_in(g, 0, 0)
        for jj in range(G):
            slot = jj % 2
            wait_in(slot)
            if jj >= 1:
                wait_sc(1 - slot)
            if jj < G - 1:
                issue_in(g, jj + 1, 1 - slot)
            compute(jj, slot)
            pltpu.async_copy(buf_a.at[slot], acc.at[idx_d.at[jj]],
                             sem_s.at[slot], add=True)
        wait_sc((G - 1) % 2)
        return carry

    lax.fori_loop(0, NG, group, 0)
    plsc.subcore_barrier()
    pltpu.sync_copy(acc.at[pl.ds(r0, RPT)], out.at[ci, pl.ds(r0, RPT)])


@functools.partial(
    pl.kernel,
    out_type=jax.ShapeDtypeStruct((2, NPAD, F), F32),
    mesh=_mesh,
    scratch_types=[
        pltpu.VMEM((G, K), jnp.int32),
        pltpu.VMEM((K, F), F32),
        pltpu.VMEM_SHARED((NPAD, F), F32),
    ],
)
def _deg_pass(ones, dstw, zz16, out, idx_d, buf1, acc):
    ci = lax.axis_index("c")
    si = lax.axis_index("s")
    w = si * 2 + ci
    r0 = si * RPT
    pltpu.sync_copy(zz16.at[pl.ds(r0, RPT)], acc.at[pl.ds(r0, RPT)])
    pltpu.sync_copy(ones, buf1)
    plsc.subcore_barrier()

    def group(g, carry):
        pltpu.sync_copy(dstw.at[w, pl.ds(g * G, G)], idx_d)

        def chunk(j, c1):
            pltpu.sync_copy(buf1, acc.at[idx_d.at[j]], add=True)
            return c1

        lax.fori_loop(0, G, chunk, 0)
        return carry

    lax.fori_loop(0, NG, group, 0)
    plsc.subcore_barrier()
    pltpu.sync_copy(acc.at[pl.ds(r0, RPT)], out.at[ci, pl.ds(r0, RPT)])


def _afn_of(x2, valid):
    x2 = jnp.where(valid, x2, 0.0)
    mean = jnp.sum(x2, axis=0, keepdims=True) * (1.0 / N)
    xc = jnp.where(valid, x2 - mean, 0.0)
    nrm = jnp.sqrt(jnp.sum(xc * xc, axis=1, keepdims=True))
    mn = jnp.sum(nrm) * (1.0 / N)
    return xc / mn


def _afn0_body(x_ref, o_ref):
    rows = lax.broadcasted_iota(jnp.int32, (NPAD, 1), 0)
    o_ref[...] = _afn_of(x_ref[...], rows < N)


def _tail_body(x_ref, h_ref, dw_ref, db_ref, o_ref):
    rows = lax.broadcasted_iota(jnp.int32, (NPAD, 1), 0)
    x2 = x_ref[...] + jnp.dot(h_ref[...], dw_ref[...],
                              preferred_element_type=F32) + db_ref[...]
    o_ref[...] = _afn_of(x2, rows < N)


def _head_body(x_ref, w_ref, b_ref, o_ref):
    o_ref[...] = jnp.maximum(
        jnp.dot(x_ref[...], w_ref[...], preferred_element_type=F32)
        + b_ref[...], 0.0)


def _pre_body(h_ref, wa_ref, wb_ref, b1_ref, ha_ref, hb_ref):
    h = h_ref[...]
    ha_ref[...] = jnp.dot(h, wa_ref[...], preferred_element_type=F32)
    hb_ref[...] = jnp.dot(h, wb_ref[...],
                          preferred_element_type=F32) + b1_ref[...]


def _post_body(o2_ref, rd_ref, h_ref, w2_ref, b2_ref, wu1_ref, wu2_ref,
               bu_ref, o_ref):
    aggpre = o2_ref[0] + o2_ref[1]
    agg = jnp.dot(aggpre, w2_ref[...],
                  preferred_element_type=F32) * rd_ref[...] + b2_ref[...]
    o_ref[...] = jnp.maximum(
        jnp.dot(h_ref[...], wu1_ref[...], preferred_element_type=F32)
        + jnp.dot(agg, wu2_ref[...], preferred_element_type=F32)
        + bu_ref[...], 0.0)


def _degfin_body(d2_ref, o_ref):
    d0 = d2_ref[0][:, 0:1] + d2_ref[1][:, 0:1]
    o_ref[...] = jnp.broadcast_to(1.0 / jnp.maximum(d0, 1.0), (NPAD, F))


def _tc(body, out_shape):
    return pl.pallas_call(body, out_shape=out_shape)

_afn0 = _tc(_afn0_body, jax.ShapeDtypeStruct((NPAD, 3), F32))
_tail = _tc(_tail_body, jax.ShapeDtypeStruct((NPAD, 3), F32))
_head = _tc(_head_body, jax.ShapeDtypeStruct((NPAD, F), F32))
_pre = _tc(_pre_body, (jax.ShapeDtypeStruct((NPAD, F), F32),
                       jax.ShapeDtypeStruct((NPAD, F), F32)))
_post = _tc(_post_body, jax.ShapeDtypeStruct((NPAD, F), F32))
_degfin = _tc(_degfin_body, jax.ShapeDtypeStruct((NPAD, F), F32))


def kernel(inds, t_rel, t_gt, t_init, params):
    edge_index = inds[0]
    edge_attr = t_rel[0]
    src = jnp.concatenate([edge_index[0], edge_index[1]])
    dst = jnp.concatenate([edge_index[1], edge_index[0]])
    attr = jnp.concatenate([edge_attr, -edge_attr], axis=0)

    pad_n = E2P - 2 * E
    pad_idx = N + (jnp.arange(pad_n, dtype=jnp.int32) % (NPAD - N))
    srcw = jnp.concatenate([src.astype(jnp.int32), pad_idx]).reshape(NW, CH, K)
    dstw = jnp.concatenate([dst.astype(jnp.int32), pad_idx]).reshape(NW, CH, K)
    attrp = jnp.concatenate([attr, jnp.zeros((pad_n, 3), F32)], axis=0)
    th = attrp.reshape(NW, CH, K, 3).transpose(0, 1, 3, 2).reshape(
        NW, 3 * CH, K)

    zz = jnp.zeros((NPAD, F), F32)
    ones = jnp.ones((K, F), F32)

    x = jnp.zeros((NPAD, 3), F32).at[:N].set(t_init[0])
    x = _afn0(x)

    d2 = _deg_pass(ones, dstw, zz)
    rec_deg = _degfin(d2)

    for i in range(2):
        p = params['mpnn%d' % i]
        enc_b = p['enc_b'].reshape(1, F)
        dec_b = p['dec_b'].reshape(1, 3)
        for it in range(2):
            h = _head(x, p['enc_W'], enc_b)
            for l in range(2):
                w1 = p['msg1_W_%d' % l]
                b1 = p['msg1_b_%d' % l].reshape(1, F)
                ha, hb = _pre(h, w1[:F], w1[F:2 * F], b1)
                o2 = _edge_pass(ha, hb, th, w1[2 * F:], srcw, dstw, zz)
                wu = p['upd_W_%d' % l]
                h = _post(o2, rec_deg, h, p['msg2_W_%d' % l],
                          p['msg2_b_%d' % l].reshape(1, F),
                          wu[:F], wu[F:], p['upd_b_%d' % l].reshape(1, F))
            x = _tail(x, h, p['dec_W'], dec_b)

    return x[:N]


# speedup vs baseline: 1.4122x; 1.4122x over previous
import functools

import jax
import jax.numpy as jnp
from jax import lax
from jax.experimental import pallas as pl
from jax.experimental.pallas import tpu as pltpu
from jax.experimental.pallas import tpu_sc as plsc

N = 10000
E = 320000
F = 128
NPAD = 10112
NW = 32
K = 64
G = 16
CH = 320
NG = CH // G
EPW = CH * K
E2P = NW * EPW
RPT = NPAD // 16
F32 = jnp.float32
BF16 = jnp.bfloat16
FH = F // 2

_mesh = plsc.VectorSubcoreMesh(core_axis_name="c", subcore_axis_name="s")


@functools.partial(
    pl.kernel,
    out_type=jax.ShapeDtypeStruct((2, NPAD, F), F32),
    mesh=_mesh,
    scratch_types=[
        pltpu.VMEM((G, K), jnp.int32),
        pltpu.VMEM((G, K), jnp.int32),
        pltpu.VMEM((2, K, FH), jnp.int32),
        pltpu.VMEM((2, K, FH), jnp.int32),
        pltpu.VMEM((2, K, FH), jnp.int32),
        pltpu.VMEM((2, K, F), F32),
        pltpu.VMEM_SHARED((NPAD, F), F32),
        pltpu.SemaphoreType.DMA((2,)),
        pltpu.SemaphoreType.DMA((2,)),
    ],
    compiler_params=pltpu.CompilerParams(needs_layout_passes=False,
                                         use_tc_tiling_on_sc=False),
)
def _edge_pass(ha, hb, cc, srcw, dstw, zz, out, idx_s, idx_d, buf_a, buf_b,
               buf_c, buf_o, acc, sem_i, sem_s):
    ci = lax.axis_index("c")
    si = lax.axis_index("s")
    w = si * 2 + ci
    r0 = si * RPT
    pltpu.sync_copy(zz.at[pl.ds(r0, RPT)], acc.at[pl.ds(r0, RPT)])
    plsc.subcore_barrier()

    def issue_in(g, jj, slot):
        base = w * EPW + (g * G + jj) * K
        pltpu.async_copy(ha.at[idx_s.at[jj]], buf_a.at[slot], sem_i.at[slot])
        pltpu.async_copy(hb.at[idx_d.at[jj]], buf_b.at[slot], sem_i.at[slot])
        pltpu.async_copy(cc.at[pl.ds(base, K)], buf_c.at[slot],
                         sem_i.at[slot])

    def wait_in(slot):
        for buf in (buf_a, buf_b, buf_c):
            pltpu.make_async_copy(ha.at[pl.ds(0, K)], buf.at[slot],
                                  sem_i.at[slot]).wait()

    def wait_sc(slot):
        pltpu.make_async_copy(zz.at[pl.ds(0, K)], buf_o.at[slot],
                              sem_s.at[slot]).wait()

    def b2f(xi):
        lo = plsc.bitcast(xi << 16, F32)
        hi = plsc.bitcast(xi & jnp.int32(-65536), F32)
        return lo, hi

    def compute(slot):
        def rows(rp, c2):
            for d in range(2):
                e = rp * 2 + d
                for u in range(FH // 16):
                    sl = pl.ds(16 * u, 16)
                    av = b2f(buf_a[slot, e, sl])
                    bv = b2f(buf_b[slot, e, sl])
                    cv = b2f(buf_c[slot, e, sl])
                    for p in range(2):
                        buf_o[slot, e, pl.ds(64 * p + 16 * u, 16)] = (
                            jnp.maximum(av[p] + bv[p] + cv[p], 0.0))
            return c2

        lax.fori_loop(0, K // 2, rows, 0)

    def group(g, carry):
        pltpu.sync_copy(srcw.at[w, pl.ds(g * G, G)], idx_s)
        pltpu.sync_copy(dstw.at[w, pl.ds(g * G, G)], idx_d)
        issue_in(g, 0, 0)
        for jj in range(G):
            slot = jj % 2
            wait_in(slot)
            if jj >= 1:
                wait_sc(1 - slot)
            if jj < G - 1:
                issue_in(g, jj + 1, 1 - slot)
            compute(slot)
            pltpu.async_copy(buf_o.at[slot], acc.at[idx_d.at[jj]],
                             sem_s.at[slot], add=True)
        wait_sc((G - 1) % 2)
        return carry

    lax.fori_loop(0, NG, group, 0)
    plsc.subcore_barrier()
    pltpu.sync_copy(acc.at[pl.ds(r0, RPT)], out.at[ci, pl.ds(r0, RPT)])


@functools.partial(
    pl.kernel,
    out_type=jax.ShapeDtypeStruct((2, NPAD, F), F32),
    mesh=_mesh,
    scratch_types=[
        pltpu.VMEM((G, K), jnp.int32),
        pltpu.VMEM((K, F), F32),
        pltpu.VMEM_SHARED((NPAD, F), F32),
    ],
)
def _deg_pass(ones, dstw, zz16, out, idx_d, buf1, acc):
    ci = lax.axis_index("c")
    si = lax.axis_index("s")
    w = si * 2 + ci
    r0 = si * RPT
    pltpu.sync_copy(zz16.at[pl.ds(r0, RPT)], acc.at[pl.ds(r0, RPT)])
    pltpu.sync_copy(ones, buf1)
    plsc.subcore_barrier()

    def group(g, carry):
        pltpu.sync_copy(dstw.at[w, pl.ds(g * G, G)], idx_d)

        def chunk(j, c1):
            pltpu.sync_copy(buf1, acc.at[idx_d.at[j]], add=True)
            return c1

        lax.fori_loop(0, G, chunk, 0)
        return carry

    lax.fori_loop(0, NG, group, 0)
    plsc.subcore_barrier()
    pltpu.sync_copy(acc.at[pl.ds(r0, RPT)], out.at[ci, pl.ds(r0, RPT)])


def _afn_of(x2, valid):
    x2 = jnp.where(valid, x2, 0.0)
    mean = jnp.sum(x2, axis=0, keepdims=True) * (1.0 / N)
    xc = jnp.where(valid, x2 - mean, 0.0)
    nrm = jnp.sqrt(jnp.sum(xc * xc, axis=1, keepdims=True))
    mn = jnp.sum(nrm) * (1.0 / N)
    return xc / mn


def _afn0_body(x_ref, o_ref):
    rows = lax.broadcasted_iota(jnp.int32, (NPAD, 1), 0)
    o_ref[...] = _afn_of(x_ref[...], rows < N)


def _tail_body(x_ref, h_ref, dw_ref, db_ref, o_ref):
    rows = lax.broadcasted_iota(jnp.int32, (NPAD, 1), 0)
    x2 = x_ref[...] + jnp.dot(h_ref[...], dw_ref[...],
                              preferred_element_type=F32) + db_ref[...]
    o_ref[...] = _afn_of(x2, rows < N)


def _head_body(x_ref, w_ref, b_ref, o_ref):
    o_ref[...] = jnp.maximum(
        jnp.dot(x_ref[...], w_ref[...], preferred_element_type=F32)
        + b_ref[...], 0.0)


def _pack32(x):
    ul = lax.bitcast_convert_type(x[:, :FH].astype(BF16),
                                  jnp.uint16).astype(jnp.uint32)
    uh = lax.bitcast_convert_type(x[:, FH:].astype(BF16),
                                  jnp.uint16).astype(jnp.uint32)
    return lax.bitcast_convert_type(ul | (uh << 16), jnp.int32)


def _pre_body(h_ref, wa_ref, wb_ref, b1_ref, ha_ref, hb_ref):
    h = h_ref[...]
    ha_ref[...] = _pack32(jnp.dot(h, wa_ref[...], preferred_element_type=F32))
    hb_ref[...] = _pack32(jnp.dot(h, wb_ref[...], preferred_element_type=F32)
                          + b1_ref[...])


_CBLK = 8192


def _cmul_body(a_ref, w_ref, o_ref):
    o_ref[...] = _pack32(jnp.dot(a_ref[...], w_ref[...],
                                 preferred_element_type=F32))


def _post_body(o2_ref, rd_ref, h_ref, w2_ref, b2_ref, wu1_ref, wu2_ref,
               bu_ref, o_ref):
    aggpre = o2_ref[0] + o2_ref[1]
    agg = jnp.dot(aggpre, w2_ref[...],
                  preferred_element_type=F32) * rd_ref[...] + b2_ref[...]
    o_ref[...] = jnp.maximum(
        jnp.dot(h_ref[...], wu1_ref[...], preferred_element_type=F32)
        + jnp.dot(agg, wu2_ref[...], preferred_element_type=F32)
        + bu_ref[...], 0.0)


def _degfin_body(d2_ref, o_ref):
    d0 = d2_ref[0][:, 0:1] + d2_ref[1][:, 0:1]
    o_ref[...] = jnp.broadcast_to(1.0 / jnp.maximum(d0, 1.0), (NPAD, F))


def _tc(body, out_shape):
    return pl.pallas_call(body, out_shape=out_shape)

_afn0 = _tc(_afn0_body, jax.ShapeDtypeStruct((NPAD, 3), F32))
_tail = _tc(_tail_body, jax.ShapeDtypeStruct((NPAD, 3), F32))
_head = _tc(_head_body, jax.ShapeDtypeStruct((NPAD, F), F32))
_pre = _tc(_pre_body, (jax.ShapeDtypeStruct((NPAD, FH), jnp.int32),
                       jax.ShapeDtypeStruct((NPAD, FH), jnp.int32)))
_cmul = pl.pallas_call(
    _cmul_body,
    grid=(E2P // _CBLK,),
    in_specs=[pl.BlockSpec((_CBLK, 3), lambda i: (i, 0)),
              pl.BlockSpec((3, F), lambda i: (0, 0))],
    out_specs=pl.BlockSpec((_CBLK, FH), lambda i: (i, 0)),
    out_shape=jax.ShapeDtypeStruct((E2P, FH), jnp.int32),
)
_post = _tc(_post_body, jax.ShapeDtypeStruct((NPAD, F), F32))
_degfin = _tc(_degfin_body, jax.ShapeDtypeStruct((NPAD, F), F32))


def kernel(inds, t_rel, t_gt, t_init, params):
    edge_index = inds[0]
    edge_attr = t_rel[0]
    src = jnp.concatenate([edge_index[0], edge_index[1]])
    dst = jnp.concatenate([edge_index[1], edge_index[0]])
    attr = jnp.concatenate([edge_attr, -edge_attr], axis=0)

    pad_n = E2P - 2 * E
    pad_idx = N + (jnp.arange(pad_n, dtype=jnp.int32) % (NPAD - N))
    srcw = jnp.concatenate([src.astype(jnp.int32), pad_idx]).reshape(NW, CH, K)
    dstw = jnp.concatenate([dst.astype(jnp.int32), pad_idx]).reshape(NW, CH, K)
    attrp = jnp.concatenate([attr, jnp.zeros((pad_n, 3), F32)], axis=0)

    zz = jnp.zeros((NPAD, F), F32)
    ones = jnp.ones((K, F), F32)

    x = jnp.zeros((NPAD, 3), F32).at[:N].set(t_init[0])
    x = _afn0(x)

    d2 = _deg_pass(ones, dstw, zz)
    rec_deg = _degfin(d2)

    cs = {}
    for i in range(2):
        p = params['mpnn%d' % i]
        for l in range(2):
            cs[(i, l)] = _cmul(attrp, p['msg1_W_%d' % l][2 * F:])

    for i in range(2):
        p = params['mpnn%d' % i]
        enc_b = p['enc_b'].reshape(1, F)
        dec_b = p['dec_b'].reshape(1, 3)
        for it in range(2):
            h = _head(x, p['enc_W'], enc_b)
            for l in range(2):
                w1 = p['msg1_W_%d' % l]
                b1 = p['msg1_b_%d' % l].reshape(1, F)
                ha, hb = _pre(h, w1[:F], w1[F:2 * F], b1)
                o2 = _edge_pass(ha, hb, cs[(i, l)], srcw, dstw, zz)
                wu = p['upd_W_%d' % l]
                h = _post(o2, rec_deg, h, p['msg2_W_%d' % l],
                          p['msg2_b_%d' % l].reshape(1, F),
                          wu[:F], wu[F:], p['upd_b_%d' % l].reshape(1, F))
            x = _tail(x, h, p['dec_W'], dec_b)

    return x[:N]


# speedup vs baseline: 1.5037x; 1.0648x over previous
import functools

import jax
import jax.numpy as jnp
from jax import lax
from jax.experimental import pallas as pl
from jax.experimental.pallas import tpu as pltpu
from jax.experimental.pallas import tpu_sc as plsc

N = 10000
E = 320000
F = 128
NPAD = 10112
NW = 32
K = 48
G = 16
CH = 432
NG = CH // G
EPW = CH * K
E2P = NW * EPW
RPT = NPAD // 16
F32 = jnp.float32
BF16 = jnp.bfloat16
FH = F // 2

_mesh = plsc.VectorSubcoreMesh(core_axis_name="c", subcore_axis_name="s")


@functools.partial(
    pl.kernel,
    out_type=jax.ShapeDtypeStruct((2, NPAD, F), F32),
    mesh=_mesh,
    scratch_types=[
        pltpu.VMEM((G, K), jnp.int32),
        pltpu.VMEM((G, K), jnp.int32),
        pltpu.VMEM((3, K, FH), jnp.int32),
        pltpu.VMEM((3, K, FH), jnp.int32),
        pltpu.VMEM((3, K, FH), jnp.int32),
        pltpu.VMEM((2, K, F), F32),
        pltpu.VMEM_SHARED((NPAD, F), F32),
        pltpu.SemaphoreType.DMA((3,)),
        pltpu.SemaphoreType.DMA((2,)),
    ],
    compiler_params=pltpu.CompilerParams(needs_layout_passes=False,
                                         use_tc_tiling_on_sc=False),
)
def _edge_pass(ha, hb, cc, srcw, dstw, zz, out, idx_s, idx_d, buf_a, buf_b,
               buf_c, buf_o, acc, sem_i, sem_s):
    ci = lax.axis_index("c")
    si = lax.axis_index("s")
    w = si * 2 + ci
    r0 = si * RPT
    pltpu.sync_copy(zz.at[pl.ds(r0, RPT)], acc.at[pl.ds(r0, RPT)])
    plsc.subcore_barrier()

    def issue_in(g, jj, slot):
        base = w * EPW + (g * G + jj) * K
        pltpu.async_copy(ha.at[idx_s.at[jj]], buf_a.at[slot], sem_i.at[slot])
        pltpu.async_copy(hb.at[idx_d.at[jj]], buf_b.at[slot], sem_i.at[slot])
        pltpu.async_copy(cc.at[pl.ds(base, K)], buf_c.at[slot],
                         sem_i.at[slot])

    def wait_in(slot):
        for buf in (buf_a, buf_b, buf_c):
            pltpu.make_async_copy(ha.at[pl.ds(0, K)], buf.at[slot],
                                  sem_i.at[slot]).wait()

    def wait_sc(slot):
        pltpu.make_async_copy(zz.at[pl.ds(0, K)], buf_o.at[slot],
                              sem_s.at[slot]).wait()

    def b2f(xi):
        lo = plsc.bitcast(xi << 16, F32)
        hi = plsc.bitcast(xi & jnp.int32(-65536), F32)
        return lo, hi

    def compute(slot, oslot):
        def rows(rp, c2):
            for d in range(2):
                e = rp * 2 + d
                for u in range(FH // 16):
                    sl = pl.ds(16 * u, 16)
                    av = b2f(buf_a[slot, e, sl])
                    bv = b2f(buf_b[slot, e, sl])
                    cv = b2f(buf_c[slot, e, sl])
                    for p in range(2):
                        buf_o[oslot, e, pl.ds(64 * p + 16 * u, 16)] = (
                            jnp.maximum(av[p] + bv[p] + cv[p], 0.0))
            return c2

        lax.fori_loop(0, K // 2, rows, 0)

    def group(g, carry):
        pltpu.sync_copy(srcw.at[w, pl.ds(g * G, G)], idx_s)
        pltpu.sync_copy(dstw.at[w, pl.ds(g * G, G)], idx_d)
        issue_in(g, 0, 0)
        issue_in(g, 1, 1)
        for jj in range(G):
            slot = jj % 3
            oslot = jj % 2
            if jj + 2 <= G - 1:
                issue_in(g, jj + 2, (jj + 2) % 3)
            wait_in(slot)
            if jj >= 2:
                wait_sc(oslot)
            compute(slot, oslot)
            pltpu.async_copy(buf_o.at[oslot], acc.at[idx_d.at[jj]],
                             sem_s.at[oslot], add=True)
        wait_sc(G % 2)
        wait_sc((G - 1) % 2)
        return carry

    lax.fori_loop(0, NG, group, 0)
    plsc.subcore_barrier()
    pltpu.sync_copy(acc.at[pl.ds(r0, RPT)], out.at[ci, pl.ds(r0, RPT)])


@functools.partial(
    pl.kernel,
    out_type=jax.ShapeDtypeStruct((2, NPAD, F), F32),
    mesh=_mesh,
    scratch_types=[
        pltpu.VMEM((G, K), jnp.int32),
        pltpu.VMEM((K, F), F32),
        pltpu.VMEM_SHARED((NPAD, F), F32),
    ],
)
def _deg_pass(ones, dstw, zz16, out, idx_d, buf1, acc):
    ci = lax.axis_index("c")
    si = lax.axis_index("s")
    w = si * 2 + ci
    r0 = si * RPT
    pltpu.sync_copy(zz16.at[pl.ds(r0, RPT)], acc.at[pl.ds(r0, RPT)])
    pltpu.sync_copy(ones, buf1)
    plsc.subcore_barrier()

    def group(g, carry):
        pltpu.sync_copy(dstw.at[w, pl.ds(g * G, G)], idx_d)

        def chunk(j, c1):
            pltpu.sync_copy(buf1, acc.at[idx_d.at[j]], add=True)
            return c1

        lax.fori_loop(0, G, chunk, 0)
        return carry

    lax.fori_loop(0, NG, group, 0)
    plsc.subcore_barrier()
    pltpu.sync_copy(acc.at[pl.ds(r0, RPT)], out.at[ci, pl.ds(r0, RPT)])


def _afn_of(x2, valid):
    x2 = jnp.where(valid, x2, 0.0)
    mean = jnp.sum(x2, axis=0, keepdims=True) * (1.0 / N)
    xc = jnp.where(valid, x2 - mean, 0.0)
    nrm = jnp.sqrt(jnp.sum(xc * xc, axis=1, keepdims=True))
    mn = jnp.sum(nrm) * (1.0 / N)
    return xc / mn


def _afn0_body(x_ref, o_ref):
    rows = lax.broadcasted_iota(jnp.int32, (NPAD, 1), 0)
    o_ref[...] = _afn_of(x_ref[...], rows < N)


def _tail_body(x_ref, h_ref, dw_ref, db_ref, o_ref):
    rows = lax.broadcasted_iota(jnp.int32, (NPAD, 1), 0)
    x2 = x_ref[...] + jnp.dot(h_ref[...], dw_ref[...],
                              preferred_element_type=F32) + db_ref[...]
    o_ref[...] = _afn_of(x2, rows < N)


def _head_body(x_ref, w_ref, b_ref, o_ref):
    o_ref[...] = jnp.maximum(
        jnp.dot(x_ref[...], w_ref[...], preferred_element_type=F32)
        + b_ref[...], 0.0)


def _pack32(x):
    ul = lax.bitcast_convert_type(x[:, :FH].astype(BF16),
                                  jnp.uint16).astype(jnp.uint32)
    uh = lax.bitcast_convert_type(x[:, FH:].astype(BF16),
                                  jnp.uint16).astype(jnp.uint32)
    return lax.bitcast_convert_type(ul | (uh << 16), jnp.int32)


def _pre_body(h_ref, wa_ref, wb_ref, b1_ref, ha_ref, hb_ref):
    h = h_ref[...]
    ha_ref[...] = _pack32(jnp.dot(h, wa_ref[...], preferred_element_type=F32))
    hb_ref[...] = _pack32(jnp.dot(h, wb_ref[...], preferred_element_type=F32)
                          + b1_ref[...])


_CBLK = 8192


def _cmul_body(a_ref, w_ref, o_ref):
    o_ref[...] = _pack32(jnp.dot(a_ref[...], w_ref[...],
                                 preferred_element_type=F32))


def _post_body(o2_ref, rd_ref, h_ref, w2_ref, b2_ref, wu1_ref, wu2_ref,
               bu_ref, o_ref):
    aggpre = o2_ref[0] + o2_ref[1]
    agg = jnp.dot(aggpre, w2_ref[...],
                  preferred_element_type=F32) * rd_ref[...] + b2_ref[...]
    o_ref[...] = jnp.maximum(
        jnp.dot(h_ref[...], wu1_ref[...], preferred_element_type=F32)
        + jnp.dot(agg, wu2_ref[...], preferred_element_type=F32)
        + bu_ref[...], 0.0)


def _degfin_body(d2_ref, o_ref):
    d0 = d2_ref[0][:, 0:1] + d2_ref[1][:, 0:1]
    o_ref[...] = jnp.broadcast_to(1.0 / jnp.maximum(d0, 1.0), (NPAD, F))


def _tc(body, out_shape):
    return pl.pallas_call(body, out_shape=out_shape)

_afn0 = _tc(_afn0_body, jax.ShapeDtypeStruct((NPAD, 3), F32))
_tail = _tc(_tail_body, jax.ShapeDtypeStruct((NPAD, 3), F32))
_head = _tc(_head_body, jax.ShapeDtypeStruct((NPAD, F), F32))
_pre = _tc(_pre_body, (jax.ShapeDtypeStruct((NPAD, FH), jnp.int32),
                       jax.ShapeDtypeStruct((NPAD, FH), jnp.int32)))
_cmul = pl.pallas_call(
    _cmul_body,
    grid=(E2P // _CBLK,),
    in_specs=[pl.BlockSpec((_CBLK, 3), lambda i: (i, 0)),
              pl.BlockSpec((3, F), lambda i: (0, 0))],
    out_specs=pl.BlockSpec((_CBLK, FH), lambda i: (i, 0)),
    out_shape=jax.ShapeDtypeStruct((E2P, FH), jnp.int32),
)
_post = _tc(_post_body, jax.ShapeDtypeStruct((NPAD, F), F32))
_degfin = _tc(_degfin_body, jax.ShapeDtypeStruct((NPAD, F), F32))


def kernel(inds, t_rel, t_gt, t_init, params):
    edge_index = inds[0]
    edge_attr = t_rel[0]
    src = jnp.concatenate([edge_index[0], edge_index[1]])
    dst = jnp.concatenate([edge_index[1], edge_index[0]])
    attr = jnp.concatenate([edge_attr, -edge_attr], axis=0)

    pad_n = E2P - 2 * E
    pad_idx = N + (jnp.arange(pad_n, dtype=jnp.int32) % (NPAD - N))
    srcw = jnp.concatenate([src.astype(jnp.int32), pad_idx]).reshape(NW, CH, K)
    dstw = jnp.concatenate([dst.astype(jnp.int32), pad_idx]).reshape(NW, CH, K)
    attrp = jnp.concatenate([attr, jnp.zeros((pad_n, 3), F32)], axis=0)

    zz = jnp.zeros((NPAD, F), F32)
    ones = jnp.ones((K, F), F32)

    x = jnp.zeros((NPAD, 3), F32).at[:N].set(t_init[0])
    x = _afn0(x)

    d2 = _deg_pass(ones, dstw, zz)
    rec_deg = _degfin(d2)

    cs = {}
    for i in range(2):
        p = params['mpnn%d' % i]
        for l in range(2):
            cs[(i, l)] = _cmul(attrp, p['msg1_W_%d' % l][2 * F:])

    for i in range(2):
        p = params['mpnn%d' % i]
        enc_b = p['enc_b'].reshape(1, F)
        dec_b = p['dec_b'].reshape(1, 3)
        for it in range(2):
            h = _head(x, p['enc_W'], enc_b)
            for l in range(2):
                w1 = p['msg1_W_%d' % l]
                b1 = p['msg1_b_%d' % l].reshape(1, F)
                ha, hb = _pre(h, w1[:F], w1[F:2 * F], b1)
                o2 = _edge_pass(ha, hb, cs[(i, l)], srcw, dstw, zz)
                wu = p['upd_W_%d' % l]
                h = _post(o2, rec_deg, h, p['msg2_W_%d' % l],
                          p['msg2_b_%d' % l].reshape(1, F),
                          wu[:F], wu[F:], p['upd_b_%d' % l].reshape(1, F))
            x = _tail(x, h, p['dec_W'], dec_b)

    return x[:N]


# speedup vs baseline: 1.8143x; 1.2066x over previous
import functools

import jax
import jax.numpy as jnp
from jax import lax
from jax.experimental import pallas as pl
from jax.experimental.pallas import tpu as pltpu
from jax.experimental.pallas import tpu_sc as plsc

N = 10000
E = 320000
F = 128
NPAD = 10112
NW = 32
K = 40
G = 16
CH = 528
NG = CH // G
EPW = CH * K
E2P = NW * EPW
RPT = NPAD // 16
F32 = jnp.float32
BF16 = jnp.bfloat16
FH = F // 2

_mesh = plsc.VectorSubcoreMesh(core_axis_name="c", subcore_axis_name="s")


@functools.partial(
    pl.kernel,
    out_type=jax.ShapeDtypeStruct((2, NPAD, F), F32),
    mesh=_mesh,
    scratch_types=[
        pltpu.VMEM((G, K), jnp.int32),
        pltpu.VMEM((G, K), jnp.int32),
        pltpu.VMEM((2, K, F), F32),
        pltpu.VMEM((2, K, F), F32),
        pltpu.VMEM((2, K, F), F32),
        pltpu.VMEM((2, K, F), F32),
        pltpu.VMEM_SHARED((NPAD, F), F32),
        pltpu.SemaphoreType.DMA((2,)),
        pltpu.SemaphoreType.DMA((2,)),
    ],
)
def _edge_pass(ha, hb, cc, srcw, dstw, zz, out, idx_s, idx_d, buf_a, buf_b,
               buf_c, buf_o, acc, sem_i, sem_s):
    ci = lax.axis_index("c")
    si = lax.axis_index("s")
    w = si * 2 + ci
    r0 = si * RPT
    pltpu.sync_copy(zz.at[pl.ds(r0, RPT)], acc.at[pl.ds(r0, RPT)])
    plsc.subcore_barrier()

    def issue_in(g, jj, slot):
        base = w * EPW + (g * G + jj) * K
        pltpu.async_copy(ha.at[idx_s.at[jj]], buf_a.at[slot], sem_i.at[slot])
        pltpu.async_copy(hb.at[idx_d.at[jj]], buf_b.at[slot], sem_i.at[slot])
        pltpu.async_copy(cc.at[pl.ds(base, K)], buf_c.at[slot],
                         sem_i.at[slot])

    def wait_in(slot):
        for buf in (buf_a, buf_b, buf_c):
            pltpu.make_async_copy(ha.at[pl.ds(0, K)], buf.at[slot],
                                  sem_i.at[slot]).wait()

    def wait_sc(slot):
        pltpu.make_async_copy(zz.at[pl.ds(0, K)], buf_o.at[slot],
                              sem_s.at[slot]).wait()

    def compute(slot):
        def rows(rp, c2):
            for d in range(2):
                e = rp * 2 + d
                for v in range(F // 16):
                    sl = pl.ds(16 * v, 16)
                    buf_o[slot, e, sl] = jnp.maximum(
                        buf_a[slot, e, sl] + buf_b[slot, e, sl]
                        + buf_c[slot, e, sl], 0.0)
            return c2

        lax.fori_loop(0, K // 2, rows, 0)

    def group(g, carry):
        pltpu.sync_copy(srcw.at[w, pl.ds(g * G, G)], idx_s)
        pltpu.sync_copy(dstw.at[w, pl.ds(g * G, G)], idx_d)
        issue_in(g, 0, 0)
        for jj in range(G):
            slot = jj % 2
            if jj + 1 <= G - 1:
                issue_in(g, jj + 1, 1 - slot)
            wait_in(slot)
            if jj >= 2:
                wait_sc(slot)
            compute(slot)
            pltpu.async_copy(buf_o.at[slot], acc.at[idx_d.at[jj]],
                             sem_s.at[slot], add=True)
        wait_sc(G % 2)
        wait_sc((G - 1) % 2)
        return carry

    lax.fori_loop(0, NG, group, 0)
    plsc.subcore_barrier()
    pltpu.sync_copy(acc.at[pl.ds(r0, RPT)], out.at[ci, pl.ds(r0, RPT)])


@functools.partial(
    pl.kernel,
    out_type=jax.ShapeDtypeStruct((2, NPAD, F), F32),
    mesh=_mesh,
    scratch_types=[
        pltpu.VMEM((G, K), jnp.int32),
        pltpu.VMEM((K, F), F32),
        pltpu.VMEM_SHARED((NPAD, F), F32),
    ],
)
def _deg_pass(ones, dstw, zz16, out, idx_d, buf1, acc):
    ci = lax.axis_index("c")
    si = lax.axis_index("s")
    w = si * 2 + ci
    r0 = si * RPT
    pltpu.sync_copy(zz16.at[pl.ds(r0, RPT)], acc.at[pl.ds(r0, RPT)])
    pltpu.sync_copy(ones, buf1)
    plsc.subcore_barrier()

    def group(g, carry):
        pltpu.sync_copy(dstw.at[w, pl.ds(g * G, G)], idx_d)

        def chunk(j, c1):
            pltpu.sync_copy(buf1, acc.at[idx_d.at[j]], add=True)
            return c1

        lax.fori_loop(0, G, chunk, 0)
        return carry

    lax.fori_loop(0, NG, group, 0)
    plsc.subcore_barrier()
    pltpu.sync_copy(acc.at[pl.ds(r0, RPT)], out.at[ci, pl.ds(r0, RPT)])


def _afn_of(x2, valid):
    x2 = jnp.where(valid, x2, 0.0)
    mean = jnp.sum(x2, axis=0, keepdims=True) * (1.0 / N)
    xc = jnp.where(valid, x2 - mean, 0.0)
    nrm = jnp.sqrt(jnp.sum(xc * xc, axis=1, keepdims=True))
    mn = jnp.sum(nrm) * (1.0 / N)
    return xc / mn


def _afn0_body(x_ref, o_ref):
    rows = lax.broadcasted_iota(jnp.int32, (NPAD, 1), 0)
    o_ref[...] = _afn_of(x_ref[...], rows < N)


def _tail_body(x_ref, h_ref, dw_ref, db_ref, o_ref):
    rows = lax.broadcasted_iota(jnp.int32, (NPAD, 1), 0)
    x2 = x_ref[...] + jnp.dot(h_ref[...], dw_ref[...],
                              preferred_element_type=F32) + db_ref[...]
    o_ref[...] = _afn_of(x2, rows < N)


def _head_body(x_ref, w_ref, b_ref, o_ref):
    o_ref[...] = jnp.maximum(
        jnp.dot(x_ref[...], w_ref[...], preferred_element_type=F32)
        + b_ref[...], 0.0)


def _pre_body(h_ref, wa_ref, wb_ref, b1_ref, ha_ref, hb_ref):
    h = h_ref[...]
    ha_ref[...] = jnp.dot(h, wa_ref[...], preferred_element_type=F32)
    hb_ref[...] = jnp.dot(h, wb_ref[...],
                          preferred_element_type=F32) + b1_ref[...]


_CBLK = 4096


def _cmul_body(a_ref, w_ref, o_ref):
    o_ref[...] = jnp.dot(a_ref[...], w_ref[...], preferred_element_type=F32)


def _post_body(o2_ref, rd_ref, h_ref, w2_ref, b2_ref, wu1_ref, wu2_ref,
               bu_ref, o_ref):
    aggpre = o2_ref[0] + o2_ref[1]
    agg = jnp.dot(aggpre, w2_ref[...],
                  preferred_element_type=F32) * rd_ref[...] + b2_ref[...]
    o_ref[...] = jnp.maximum(
        jnp.dot(h_ref[...], wu1_ref[...], preferred_element_type=F32)
        + jnp.dot(agg, wu2_ref[...], preferred_element_type=F32)
        + bu_ref[...], 0.0)


def _degfin_body(d2_ref, o_ref):
    d0 = d2_ref[0][:, 0:1] + d2_ref[1][:, 0:1]
    o_ref[...] = jnp.broadcast_to(1.0 / jnp.maximum(d0, 1.0), (NPAD, F))


def _tc(body, out_shape):
    return pl.pallas_call(body, out_shape=out_shape)

_afn0 = _tc(_afn0_body, jax.ShapeDtypeStruct((NPAD, 3), F32))
_tail = _tc(_tail_body, jax.ShapeDtypeStruct((NPAD, 3), F32))
_head = _tc(_head_body, jax.ShapeDtypeStruct((NPAD, F), F32))
_pre = _tc(_pre_body, (jax.ShapeDtypeStruct((NPAD, F), F32),
                       jax.ShapeDtypeStruct((NPAD, F), F32)))
_cmul = pl.pallas_call(
    _cmul_body,
    grid=(E2P // _CBLK,),
    in_specs=[pl.BlockSpec((_CBLK, 3), lambda i: (i, 0)),
              pl.BlockSpec((3, F), lambda i: (0, 0))],
    out_specs=pl.BlockSpec((_CBLK, F), lambda i: (i, 0)),
    out_shape=jax.ShapeDtypeStruct((E2P, F), F32),
)
_post = _tc(_post_body, jax.ShapeDtypeStruct((NPAD, F), F32))
_degfin = _tc(_degfin_body, jax.ShapeDtypeStruct((NPAD, F), F32))


def kernel(inds, t_rel, t_gt, t_init, params):
    edge_index = inds[0]
    edge_attr = t_rel[0]
    src = jnp.concatenate([edge_index[0], edge_index[1]])
    dst = jnp.concatenate([edge_index[1], edge_index[0]])
    attr = jnp.concatenate([edge_attr, -edge_attr], axis=0)

    pad_n = E2P - 2 * E
    pad_idx = N + (jnp.arange(pad_n, dtype=jnp.int32) % (NPAD - N))
    srcw = jnp.concatenate([src.astype(jnp.int32), pad_idx]).reshape(NW, CH, K)
    dstw = jnp.concatenate([dst.astype(jnp.int32), pad_idx]).reshape(NW, CH, K)
    attrp = jnp.concatenate([attr, jnp.zeros((pad_n, 3), F32)], axis=0)

    zz = jnp.zeros((NPAD, F), F32)
    ones = jnp.ones((K, F), F32)

    x = jnp.zeros((NPAD, 3), F32).at[:N].set(t_init[0])
    x = _afn0(x)

    d2 = _deg_pass(ones, dstw, zz)
    rec_deg = _degfin(d2)

    cs = {}
    for i in range(2):
        p = params['mpnn%d' % i]
        for l in range(2):
            cs[(i, l)] = _cmul(attrp, p['msg1_W_%d' % l][2 * F:])

    for i in range(2):
        p = params['mpnn%d' % i]
        enc_b = p['enc_b'].reshape(1, F)
        dec_b = p['dec_b'].reshape(1, 3)
        for it in range(2):
            h = _head(x, p['enc_W'], enc_b)
            for l in range(2):
                w1 = p['msg1_W_%d' % l]
                b1 = p['msg1_b_%d' % l].reshape(1, F)
                ha, hb = _pre(h, w1[:F], w1[F:2 * F], b1)
                o2 = _edge_pass(ha, hb, cs[(i, l)], srcw, dstw, zz)
                wu = p['upd_W_%d' % l]
                h = _post(o2, rec_deg, h, p['msg2_W_%d' % l],
                          p['msg2_b_%d' % l].reshape(1, F),
                          wu[:F], wu[F:], p['upd_b_%d' % l].reshape(1, F))
            x = _tail(x, h, p['dec_W'], dec_b)

    return x[:N]


# speedup vs baseline: 1.9631x; 1.0820x over previous
import functools

import jax
import jax.numpy as jnp
from jax import lax
from jax.experimental import pallas as pl
from jax.experimental.pallas import tpu as pltpu
from jax.experimental.pallas import tpu_sc as plsc

N = 10000
E = 320000
F = 128
NPAD = 10112
NW = 32
K = 40
G = 32
CH = 512
NG = CH // G
EPW = CH * K
E2P = NW * EPW
RPT = NPAD // 16
F32 = jnp.float32
BF16 = jnp.bfloat16
FH = F // 2

_mesh = plsc.VectorSubcoreMesh(core_axis_name="c", subcore_axis_name="s")


@functools.partial(
    pl.kernel,
    out_type=jax.ShapeDtypeStruct((2, NPAD, F), F32),
    mesh=_mesh,
    scratch_types=[
        pltpu.VMEM((G, K), jnp.int32),
        pltpu.VMEM((G, K), jnp.int32),
        pltpu.VMEM((2, K, F), F32),
        pltpu.VMEM((2, K, F), F32),
        pltpu.VMEM((2, K, F), F32),
        pltpu.VMEM((2, K, F), F32),
        pltpu.VMEM_SHARED((NPAD, F), F32),
        pltpu.SemaphoreType.DMA((2,)),
        pltpu.SemaphoreType.DMA((2,)),
    ],
)
def _edge_pass(ha, hb, cc, srcw, dstw, zz, out, idx_s, idx_d, buf_a, buf_b,
               buf_c, buf_o, acc, sem_i, sem_s):
    ci = lax.axis_index("c")
    si = lax.axis_index("s")
    w = si * 2 + ci
    r0 = si * RPT
    pltpu.sync_copy(zz.at[pl.ds(r0, RPT)], acc.at[pl.ds(r0, RPT)])
    plsc.subcore_barrier()

    def issue_in(g, jj, slot):
        base = w * EPW + (g * G + jj) * K
        pltpu.async_copy(ha.at[idx_s.at[jj]], buf_a.at[slot], sem_i.at[slot])
        pltpu.async_copy(hb.at[idx_d.at[jj]], buf_b.at[slot], sem_i.at[slot])
        pltpu.async_copy(cc.at[pl.ds(base, K)], buf_c.at[slot],
                         sem_i.at[slot])

    def wait_in(slot):
        for buf in (buf_a, buf_b, buf_c):
            pltpu.make_async_copy(ha.at[pl.ds(0, K)], buf.at[slot],
                                  sem_i.at[slot]).wait()

    def wait_sc(slot):
        pltpu.make_async_copy(zz.at[pl.ds(0, K)], buf_o.at[slot],
                              sem_s.at[slot]).wait()

    def compute(slot):
        def rows(rp, c2):
            for d in range(2):
                e = rp * 2 + d
                for v in range(F // 16):
                    sl = pl.ds(16 * v, 16)
                    buf_o[slot, e, sl] = jnp.maximum(
                        buf_a[slot, e, sl] + buf_b[slot, e, sl]
                        + buf_c[slot, e, sl], 0.0)
            return c2

        lax.fori_loop(0, K // 2, rows, 0)

    def group(g, carry):
        pltpu.sync_copy(srcw.at[w, pl.ds(g * G, G)], idx_s)
        pltpu.sync_copy(dstw.at[w, pl.ds(g * G, G)], idx_d)
        issue_in(g, 0, 0)
        for jj in range(G):
            slot = jj % 2
            if jj + 1 <= G - 1:
                issue_in(g, jj + 1, 1 - slot)
            wait_in(slot)
            if jj >= 2:
                wait_sc(slot)
            compute(slot)
            pltpu.async_copy(buf_o.at[slot], acc.at[idx_d.at[jj]],
                             sem_s.at[slot], add=True)
        wait_sc(G % 2)
        wait_sc((G - 1) % 2)
        return carry

    lax.fori_loop(0, NG, group, 0)
    plsc.subcore_barrier()
    pltpu.sync_copy(acc.at[pl.ds(r0, RPT)], out.at[ci, pl.ds(r0, RPT)])


@functools.partial(
    pl.kernel,
    out_type=jax.ShapeDtypeStruct((2, NPAD, F), F32),
    mesh=_mesh,
    scratch_types=[
        pltpu.VMEM((G, K), jnp.int32),
        pltpu.VMEM((K, F), F32),
        pltpu.VMEM_SHARED((NPAD, F), F32),
    ],
)
def _deg_pass(ones, dstw, zz16, out, idx_d, buf1, acc):
    ci = lax.axis_index("c")
    si = lax.axis_index("s")
    w = si * 2 + ci
    r0 = si * RPT
    pltpu.sync_copy(zz16.at[pl.ds(r0, RPT)], acc.at[pl.ds(r0, RPT)])
    pltpu.sync_copy(ones, buf1)
    plsc.subcore_barrier()

    def group(g, carry):
        pltpu.sync_copy(dstw.at[w, pl.ds(g * G, G)], idx_d)

        def chunk(j, c1):
            pltpu.sync_copy(buf1, acc.at[idx_d.at[j]], add=True)
            return c1

        lax.fori_loop(0, G, chunk, 0)
        return carry

    lax.fori_loop(0, NG, group, 0)
    plsc.subcore_barrier()
    pltpu.sync_copy(acc.at[pl.ds(r0, RPT)], out.at[ci, pl.ds(r0, RPT)])


def _afn_of(x2, valid):
    x2 = jnp.where(valid, x2, 0.0)
    mean = jnp.sum(x2, axis=0, keepdims=True) * (1.0 / N)
    xc = jnp.where(valid, x2 - mean, 0.0)
    nrm = jnp.sqrt(jnp.sum(xc * xc, axis=1, keepdims=True))
    mn = jnp.sum(nrm) * (1.0 / N)
    return xc / mn


def _afn0_body(x_ref, o_ref):
    rows = lax.broadcasted_iota(jnp.int32, (NPAD, 1), 0)
    o_ref[...] = _afn_of(x_ref[...], rows < N)


def _tail_body(x_ref, h_ref, dw_ref, db_ref, o_ref):
    rows = lax.broadcasted_iota(jnp.int32, (NPAD, 1), 0)
    x2 = x_ref[...] + jnp.dot(h_ref[...], dw_ref[...],
                              preferred_element_type=F32) + db_ref[...]
    o_ref[...] = _afn_of(x2, rows < N)


def _head_body(x_ref, w_ref, b_ref, o_ref):
    o_ref[...] = jnp.maximum(
        jnp.dot(x_ref[...], w_ref[...], preferred_element_type=F32)
        + b_ref[...], 0.0)


def _pre_body(h_ref, wa_ref, wb_ref, b1_ref, ha_ref, hb_ref):
    h = h_ref[...]
    ha_ref[...] = jnp.dot(h, wa_ref[...], preferred_element_type=F32)
    hb_ref[...] = jnp.dot(h, wb_ref[...],
                          preferred_element_type=F32) + b1_ref[...]


_CBLK = 4096


def _cmul_body(a_ref, w_ref, o_ref):
    o_ref[...] = jnp.dot(a_ref[...], w_ref[...], preferred_element_type=F32)


def _post_body(o2_ref, rd_ref, h_ref, w2_ref, b2_ref, wu1_ref, wu2_ref,
               bu_ref, o_ref):
    aggpre = o2_ref[0] + o2_ref[1]
    agg = jnp.dot(aggpre, w2_ref[...],
                  preferred_element_type=F32) * rd_ref[...] + b2_ref[...]
    o_ref[...] = jnp.maximum(
        jnp.dot(h_ref[...], wu1_ref[...], preferred_element_type=F32)
        + jnp.dot(agg, wu2_ref[...], preferred_element_type=F32)
        + bu_ref[...], 0.0)


def _degfin_body(d2_ref, o_ref):
    d0 = d2_ref[0][:, 0:1] + d2_ref[1][:, 0:1]
    o_ref[...] = jnp.broadcast_to(1.0 / jnp.maximum(d0, 1.0), (NPAD, F))


def _tc(body, out_shape):
    return pl.pallas_call(body, out_shape=out_shape)

_afn0 = _tc(_afn0_body, jax.ShapeDtypeStruct((NPAD, 3), F32))
_tail = _tc(_tail_body, jax.ShapeDtypeStruct((NPAD, 3), F32))
_head = _tc(_head_body, jax.ShapeDtypeStruct((NPAD, F), F32))
_pre = _tc(_pre_body, (jax.ShapeDtypeStruct((NPAD, F), F32),
                       jax.ShapeDtypeStruct((NPAD, F), F32)))
_cmul = pl.pallas_call(
    _cmul_body,
    grid=(E2P // _CBLK,),
    in_specs=[pl.BlockSpec((_CBLK, 3), lambda i: (i, 0)),
              pl.BlockSpec((3, F), lambda i: (0, 0))],
    out_specs=pl.BlockSpec((_CBLK, F), lambda i: (i, 0)),
    out_shape=jax.ShapeDtypeStruct((E2P, F), F32),
)
_post = _tc(_post_body, jax.ShapeDtypeStruct((NPAD, F), F32))
_degfin = _tc(_degfin_body, jax.ShapeDtypeStruct((NPAD, F), F32))


def kernel(inds, t_rel, t_gt, t_init, params):
    edge_index = inds[0]
    edge_attr = t_rel[0]
    src = jnp.concatenate([edge_index[0], edge_index[1]])
    dst = jnp.concatenate([edge_index[1], edge_index[0]])
    attr = jnp.concatenate([edge_attr, -edge_attr], axis=0)

    pad_n = E2P - 2 * E
    pad_idx = N + (jnp.arange(pad_n, dtype=jnp.int32) % (NPAD - N))
    srcw = jnp.concatenate([src.astype(jnp.int32), pad_idx]).reshape(NW, CH, K)
    dstw = jnp.concatenate([dst.astype(jnp.int32), pad_idx]).reshape(NW, CH, K)
    attrp = jnp.concatenate([attr, jnp.zeros((pad_n, 3), F32)], axis=0)

    zz = jnp.zeros((NPAD, F), F32)
    ones = jnp.ones((K, F), F32)

    x = jnp.zeros((NPAD, 3), F32).at[:N].set(t_init[0])
    x = _afn0(x)

    d2 = _deg_pass(ones, dstw, zz)
    rec_deg = _degfin(d2)

    cs = {}
    for i in range(2):
        p = params['mpnn%d' % i]
        for l in range(2):
            cs[(i, l)] = _cmul(attrp, p['msg1_W_%d' % l][2 * F:])

    for i in range(2):
        p = params['mpnn%d' % i]
        enc_b = p['enc_b'].reshape(1, F)
        dec_b = p['dec_b'].reshape(1, 3)
        for it in range(2):
            h = _head(x, p['enc_W'], enc_b)
            for l in range(2):
                w1 = p['msg1_W_%d' % l]
                b1 = p['msg1_b_%d' % l].reshape(1, F)
                ha, hb = _pre(h, w1[:F], w1[F:2 * F], b1)
                o2 = _edge_pass(ha, hb, cs[(i, l)], srcw, dstw, zz)
                wu = p['upd_W_%d' % l]
                h = _post(o2, rec_deg, h, p['msg2_W_%d' % l],
                          p['msg2_b_%d' % l].reshape(1, F),
                          wu[:F], wu[F:], p['upd_b_%d' % l].reshape(1, F))
            x = _tail(x, h, p['dec_W'], dec_b)

    return x[:N]
